# Initial kernel scaffold; baseline (speedup 1.0000x reference)
import jax
import jax.numpy as jnp
from jax.experimental import pallas as pl


def kernel(X_gene, H_gene, vertex_spa, edges_spa, W1_1, b1_1, W1_2, att_e1, W2_1, b2_1, W2_2, att_e2, attW1, attb1, attW2, att1W1, att1b1, att1W2):
    raise NotImplementedError("write your pallas kernel here")


# speedup vs baseline: 8.9725x; 8.9725x over previous
import functools

import jax
import jax.numpy as jnp
from jax import lax
from jax.experimental import pallas as pl
from jax.experimental.pallas import tpu as pltpu
from jax.experimental.pallas import tpu_sc as plsc

N = 10000
D = 128
F = 128
AUG = 144
NNZ = 160000
CHUNK = 128
NCHUNK = NNZ // CHUNK
NCHUNK_PAD = 1280
NW = 32
ROWS_PER_SUB = N // 16
NEG_SLOPE = 0.2


def _sc_body(table, sidx, didx, zer, out, acc, isrc, idst, rows, sem):
    c = lax.axis_index("c")
    s = lax.axis_index("s")
    wid = s * 2 + c

    pltpu.sync_copy(zer.at[pl.ds(s * ROWS_PER_SUB, ROWS_PER_SUB)],
                    acc.at[pl.ds(s * ROWS_PER_SUB, ROWS_PER_SUB)])
    plsc.subcore_barrier()

    start = 39 * wid + jnp.minimum(wid, 2)
    nt = jnp.where(wid < 2, 40, 39)

    pltpu.sync_copy(sidx.at[pl.ds(start, 40)], isrc)
    pltpu.sync_copy(didx.at[pl.ds(start, 40)], idst)

    def body(t, carry):
        pltpu.async_copy(table.at[isrc.at[t]], rows, sem).wait()
        pltpu.sync_copy(rows, acc.at[idst.at[t]], add=True)
        return carry

    lax.fori_loop(0, nt, body, 0)

    plsc.subcore_barrier()
    pltpu.sync_copy(acc.at[pl.ds(s * ROWS_PER_SUB, ROWS_PER_SUB)],
                    out.at[c, pl.ds(s * ROWS_PER_SUB, ROWS_PER_SUB)])


_sc_scatter = pl.kernel(
    _sc_body,
    mesh=plsc.VectorSubcoreMesh(core_axis_name="c", subcore_axis_name="s"),
    compiler_params=pltpu.CompilerParams(use_tc_tiling_on_sc=False),
    out_type=jax.ShapeDtypeStruct((2, N, AUG), jnp.float32),
    scratch_types=[
        pltpu.VMEM_SHARED((N, AUG), jnp.float32),
        pltpu.VMEM((40, CHUNK), jnp.int32),
        pltpu.VMEM((40, CHUNK), jnp.int32),
        pltpu.VMEM((CHUNK, AUG), jnp.float32),
        pltpu.SemaphoreType.DMA,
    ],
)


BM_BIG = 200


def _big_mm_body(h_ref, y_ref, o_ref):
    o_ref[...] = jnp.dot(h_ref[...], y_ref[...],
                         preferred_element_type=jnp.float32)


def _big_mm(H, Y):
    return pl.pallas_call(
        _big_mm_body,
        grid=(N // BM_BIG,),
        in_specs=[
            pl.BlockSpec((BM_BIG, N), lambda i: (i, 0)),
            pl.BlockSpec((N, F), lambda i: (0, 0)),
        ],
        out_specs=pl.BlockSpec((BM_BIG, F), lambda i: (i, 0)),
        out_shape=jax.ShapeDtypeStruct((N, F), jnp.float32),
    )(H, Y)


BM = 1000


def _prep1_body(x_ref, w1_ref, b1_ref, w2_ref, y_ref, t_ref):
    x = x_ref[...]
    y_ref[...] = jnp.dot(x, w1_ref[...], preferred_element_type=jnp.float32) \
        + b1_ref[...]
    x0 = jnp.dot(x, w2_ref[...], preferred_element_type=jnp.float32)
    t_ref[...] = jnp.concatenate(
        [x0, jnp.ones((BM, AUG - F), jnp.float32)], axis=1)


def _prep1(X, W1, b1, W2):
    return pl.pallas_call(
        _prep1_body,
        grid=(N // BM,),
        in_specs=[
            pl.BlockSpec((BM, D), lambda i: (i, 0)),
            pl.BlockSpec((D, F), lambda i: (0, 0)),
            pl.BlockSpec((1, F), lambda i: (0, 0)),
            pl.BlockSpec((D, F), lambda i: (0, 0)),
        ],
        out_specs=[
            pl.BlockSpec((BM, F), lambda i: (i, 0)),
            pl.BlockSpec((BM, AUG), lambda i: (i, 0)),
        ],
        out_shape=[
            jax.ShapeDtypeStruct((N, F), jnp.float32),
            jax.ShapeDtypeStruct((N, AUG), jnp.float32),
        ],
    )(X, W1, b1, W2)


def _edge_prep_body(c0_ref, c1_ref, att_ref, t_ref):
    a = c0_ref[...] + c1_ref[...]
    deg = jnp.maximum(a[:, F:F + 1], 1.0)
    xe = a[:, :F] / deg
    alpha = jnp.sum(xe * att_ref[...], axis=1, keepdims=True)
    alpha = jnp.where(alpha >= 0, alpha, NEG_SLOPE * alpha)
    s = jnp.exp(alpha)
    t_ref[...] = jnp.concatenate(
        [xe * s, jnp.broadcast_to(s, (BM, AUG - F))], axis=1)


def _edge_prep(A1, att):
    return pl.pallas_call(
        _edge_prep_body,
        grid=(N // BM,),
        in_specs=[
            pl.BlockSpec((BM, AUG), lambda i: (i, 0)),
            pl.BlockSpec((BM, AUG), lambda i: (i, 0)),
            pl.BlockSpec((1, F), lambda i: (0, 0)),
        ],
        out_specs=pl.BlockSpec((BM, AUG), lambda i: (i, 0)),
        out_shape=jax.ShapeDtypeStruct((N, AUG), jnp.float32),
    )(A1[0], A1[1], att)


def _pair_attn(za, zb, w1_ref, b1_ref, w2_ref):
    ta = jnp.tanh(jnp.dot(za, w1_ref[...], preferred_element_type=jnp.float32)
                  + b1_ref[...])
    tb = jnp.tanh(jnp.dot(zb, w1_ref[...], preferred_element_type=jnp.float32)
                  + b1_ref[...])
    wa = jnp.sum(ta * w2_ref[...], axis=1, keepdims=True)
    wb = jnp.sum(tb * w2_ref[...], axis=1, keepdims=True)
    m = jnp.maximum(wa, wb)
    ea = jnp.exp(wa - m)
    eb = jnp.exp(wb - m)
    inv = 1.0 / (ea + eb)
    return (ea * inv) * za + (eb * inv) * zb


def _attn_mid_body(h1_ref, c0_ref, c1_ref, aw1_ref, ab1_ref, aw2_ref,
                   w21_ref, b21_ref, w22_ref, y2_ref, t_ref):
    h1 = h1_ref[...]
    a = c0_ref[...] + c1_ref[...]
    h2 = a[:, :F] / (a[:, F:F + 1] + 1e-16)
    h3f = _pair_attn(h1, h2, aw1_ref, ab1_ref, aw2_ref)
    y2_ref[...] = jnp.dot(h3f, w21_ref[...],
                          preferred_element_type=jnp.float32) + b21_ref[...]
    x0b = jnp.dot(h1, w22_ref[...], preferred_element_type=jnp.float32)
    t_ref[...] = jnp.concatenate(
        [x0b, jnp.ones((BM, AUG - F), jnp.float32)], axis=1)


def _attn_mid(H1, A2, aW1, ab1, aW2r, W21, b21, W22):
    return pl.pallas_call(
        _attn_mid_body,
        grid=(N // BM,),
        in_specs=[
            pl.BlockSpec((BM, F), lambda i: (i, 0)),
            pl.BlockSpec((BM, AUG), lambda i: (i, 0)),
            pl.BlockSpec((BM, AUG), lambda i: (i, 0)),
            pl.BlockSpec((F, 16), lambda i: (0, 0)),
            pl.BlockSpec((1, 16), lambda i: (0, 0)),
            pl.BlockSpec((1, 16), lambda i: (0, 0)),
            pl.BlockSpec((F, F), lambda i: (0, 0)),
            pl.BlockSpec((1, F), lambda i: (0, 0)),
            pl.BlockSpec((F, F), lambda i: (0, 0)),
        ],
        out_specs=[
            pl.BlockSpec((BM, F), lambda i: (i, 0)),
            pl.BlockSpec((BM, AUG), lambda i: (i, 0)),
        ],
        out_shape=[
            jax.ShapeDtypeStruct((N, F), jnp.float32),
            jax.ShapeDtypeStruct((N, AUG), jnp.float32),
        ],
    )(H1, A2[0], A2[1], aW1, ab1, aW2r, W21, b21, W22)


def _attn_final_body(o1_ref, c0_ref, c1_ref, aw1_ref, ab1_ref, aw2_ref,
                     o2_ref, oa_ref):
    o1 = o1_ref[...]
    a = c0_ref[...] + c1_ref[...]
    o2 = a[:, :F] / (a[:, F:F + 1] + 1e-16)
    o2_ref[...] = o2
    oa_ref[...] = _pair_attn(o2, o1, aw1_ref, ab1_ref, aw2_ref)


def _attn_final(out1, A2, aW1, ab1, aW2r):
    return pl.pallas_call(
        _attn_final_body,
        grid=(N // BM,),
        in_specs=[
            pl.BlockSpec((BM, F), lambda i: (i, 0)),
            pl.BlockSpec((BM, AUG), lambda i: (i, 0)),
            pl.BlockSpec((BM, AUG), lambda i: (i, 0)),
            pl.BlockSpec((F, 16), lambda i: (0, 0)),
            pl.BlockSpec((1, 16), lambda i: (0, 0)),
            pl.BlockSpec((1, 16), lambda i: (0, 0)),
        ],
        out_specs=[
            pl.BlockSpec((BM, F), lambda i: (i, 0)),
            pl.BlockSpec((BM, F), lambda i: (i, 0)),
        ],
        out_shape=[
            jax.ShapeDtypeStruct((N, F), jnp.float32),
            jax.ShapeDtypeStruct((N, F), jnp.float32),
        ],
    )(out1, A2[0], A2[1], aW1, ab1, aW2r)


def kernel(X_gene, H_gene, vertex_spa, edges_spa, W1_1, b1_1, W1_2, att_e1,
           W2_1, b2_1, W2_2, att_e2, attW1, attb1, attW2, att1W1, att1b1,
           att1W2):
    pad = NCHUNK_PAD * CHUNK - NNZ
    v2d = jnp.pad(vertex_spa, (0, pad)).reshape(NCHUNK_PAD, CHUNK)
    e2d = jnp.pad(edges_spa, (0, pad)).reshape(NCHUNK_PAD, CHUNK)
    zer = jnp.zeros((N, AUG), jnp.float32)

    b1_1r = b1_1.reshape(1, F)
    b2_1r = b2_1.reshape(1, F)
    att1r = att_e1.reshape(1, F)
    att2r = att_e2.reshape(1, F)
    attb1r = attb1.reshape(1, 16)
    attW2r = attW2.reshape(1, 16)
    att1b1r = att1b1.reshape(1, 16)
    att1W2r = att1W2.reshape(1, 16)

    Y1, T1a = _prep1(X_gene, W1_1, b1_1r, W1_2)
    A1a = _sc_scatter(T1a, v2d, e2d, zer)
    H1 = _big_mm(H_gene, Y1)
    T2a = _edge_prep(A1a, att1r)
    A2a = _sc_scatter(T2a, e2d, v2d, zer)
    Y2, T1b = _attn_mid(H1, A2a, attW1, attb1r, attW2r, W2_1, b2_1r, W2_2)
    A1b = _sc_scatter(T1b, v2d, e2d, zer)
    out1 = _big_mm(H_gene, Y2)
    T2b = _edge_prep(A1b, att2r)
    A2b = _sc_scatter(T2b, e2d, v2d, zer)
    out2, out_atten = _attn_final(out1, A2b, att1W1, att1b1r, att1W2r)
    return out1, out2, out_atten
